# Initial kernel scaffold; baseline (speedup 1.0000x reference)
import jax
import jax.numpy as jnp
from jax.experimental import pallas as pl


def kernel(x, edge_index, edge_type, comp1, basis1, root1, bias1, comp2, basis2, root2, bias2):
    raise NotImplementedError("write your pallas kernel here")


# speedup vs baseline: 22.4552x; 22.4552x over previous
import functools

import jax
import jax.numpy as jnp
from jax import lax
from jax.experimental import pallas as pl
from jax.experimental.pallas import tpu as pltpu
from jax.experimental.pallas import tpu_sc as plsc

N = 10000
E = 320000
IN_CH = 128
HID = 64
OUT_CH = 128
R = 8
B = 4

NC = 2
NS = 16
NW = NC * NS
EW = E // NW
K = 80
NB = EW // K
RN = R * N
CNT_SL = RN // NS
ROW_SL = N // NS


def _sc_mesh():
    return plsc.VectorSubcoreMesh(
        core_axis_name="c", subcore_axis_name="s", num_cores=NC, num_subcores=NS
    )


def _sc_counts(cidx):
    BNC = 5008

    @functools.partial(
        pl.kernel,
        out_type=jax.ShapeDtypeStruct((NC * RN,), jnp.float32),
        mesh=_sc_mesh(),
        scratch_types=[
            pltpu.VMEM((K,), jnp.int32),
            pltpu.VMEM((K,), jnp.float32),
            pltpu.VMEM((BNC,), jnp.float32),
            pltpu.VMEM_SHARED((RN,), jnp.float32),
        ],
    )
    def run(cidx_hbm, out_hbm, idx_v, ones_v, bnc_v, acc_sh):
        c = lax.axis_index("c")
        s = lax.axis_index("s")
        wid = s * NC + c

        def fill_ones(i, carry):
            ones_v[pl.ds(16 * i, 16)] = jnp.ones((16,), jnp.float32)
            return carry

        lax.fori_loop(0, K // 16, fill_ones, 0)

        def fill_z(i, carry):
            bnc_v[pl.ds(16 * i, 16)] = jnp.zeros((16,), jnp.float32)
            return carry

        lax.fori_loop(0, BNC // 16, fill_z, 0)
        pltpu.sync_copy(bnc_v.at[pl.ds(0, CNT_SL)],
                        acc_sh.at[pl.ds(s * CNT_SL, CNT_SL)])
        plsc.subcore_barrier()

        def blk(b, carry):
            pltpu.sync_copy(cidx_hbm.at[pl.ds(wid * EW + b * K, K)], idx_v)
            pltpu.sync_copy(ones_v, acc_sh.at[idx_v], add=True)
            return carry

        lax.fori_loop(0, NB, blk, 0)
        plsc.subcore_barrier()
        pltpu.sync_copy(acc_sh.at[pl.ds(s * CNT_SL, CNT_SL)],
                        bnc_v.at[pl.ds(0, CNT_SL)])
        pltpu.sync_copy(bnc_v.at[pl.ds(0, CNT_SL)],
                        out_hbm.at[pl.ds(c * RN + s * CNT_SL, CNT_SL)])

    return run(cidx).reshape(NC, RN)


def _sc_edge_pass(hx_flat, inv, sidx, cidx, dst, dout):
    j16 = dout // 16
    CH = 125

    @functools.partial(
        pl.kernel,
        out_type=jax.ShapeDtypeStruct((NC * N, dout), jnp.float32),
        mesh=_sc_mesh(),
        scratch_types=[
            pltpu.VMEM((K,), jnp.int32),
            pltpu.VMEM((K,), jnp.int32),
            pltpu.VMEM((K,), jnp.int32),
            pltpu.VMEM((K,), jnp.float32),
            pltpu.VMEM((K, dout), jnp.float32),
            pltpu.VMEM((CH, dout), jnp.float32),
            pltpu.VMEM_SHARED((N, dout), jnp.float32),
            pltpu.SemaphoreType.DMA,
            pltpu.SemaphoreType.DMA,
        ],
        compiler_params=pltpu.CompilerParams(
            use_tc_tiling_on_sc=False, needs_layout_passes=False),
    )
    def run(hx_hbm, inv_hbm, sidx_hbm, cidx_hbm, dst_hbm, out_hbm,
            sidx_v, cidx_v, dst_v, w_v, rows_v, bnc_v, acc_sh, sem_a, sem_b):
        c = lax.axis_index("c")
        s = lax.axis_index("s")
        wid = s * NC + c

        def fill_z(i, carry):
            for j in range(j16):
                bnc_v[i, pl.ds(16 * j, 16)] = jnp.zeros((16,), jnp.float32)
            return carry

        lax.fori_loop(0, CH, fill_z, 0)

        def zero_chunk(q, carry):
            pltpu.sync_copy(bnc_v, acc_sh.at[pl.ds(s * ROW_SL + q * CH, CH)])
            return carry

        lax.fori_loop(0, ROW_SL // CH, zero_chunk, 0)
        plsc.subcore_barrier()

        def blk(b, carry):
            base = wid * EW + b * K
            pltpu.sync_copy(sidx_hbm.at[pl.ds(base, K)], sidx_v)
            pltpu.sync_copy(cidx_hbm.at[pl.ds(base, K)], cidx_v)
            pltpu.sync_copy(dst_hbm.at[pl.ds(base, K)], dst_v)
            ga = pltpu.async_copy(hx_hbm.at[sidx_v], rows_v, sem_a)
            gb = pltpu.async_copy(inv_hbm.at[cidx_v], w_v, sem_b)
            ga.wait()
            gb.wait()

            def edge(e, carry2):
                w = plsc.load_gather(w_v, [jnp.full((16,), e, jnp.int32)])
                for j in range(j16):
                    rows_v[e, pl.ds(16 * j, 16)] = rows_v[e, pl.ds(16 * j, 16)] * w
                return carry2

            lax.fori_loop(0, K, edge, 0)
            pltpu.sync_copy(rows_v, acc_sh.at[dst_v], add=True)
            return carry

        lax.fori_loop(0, NB, blk, 0)
        plsc.subcore_barrier()

        def out_chunk(q, carry):
            pltpu.sync_copy(acc_sh.at[pl.ds(s * ROW_SL + q * CH, CH)], bnc_v)
            pltpu.sync_copy(
                bnc_v, out_hbm.at[pl.ds(c * N + s * ROW_SL + q * CH, CH)])
            return carry

        lax.fori_loop(0, ROW_SL // CH, out_chunk, 0)

    return run(hx_flat, inv, sidx, cidx, dst).reshape(NC, N, dout)


_TN = 2000


def _tc_dense(x, comp_pad, basis, root, bias, dout):
    din = x.shape[1]

    def body(comp_ref, x_ref, basis_ref, root_ref, bias_ref, out_ref):
        r = pl.program_id(0)
        w = jnp.zeros((din, dout), jnp.float32)
        for b in range(B):
            w = w + comp_ref[r, b] * basis_ref[b]
        is_root = jnp.where(r == R, 1.0, 0.0).astype(jnp.float32)
        w = w + is_root * root_ref[...]
        acc = jnp.dot(x_ref[...], w, preferred_element_type=jnp.float32)
        out_ref[0] = acc + is_root * bias_ref[...]

    return pl.pallas_call(
        body,
        grid=(R + 1, N // _TN),
        in_specs=[
            pl.BlockSpec(memory_space=pltpu.SMEM),
            pl.BlockSpec((_TN, din), lambda r, n: (n, 0)),
            pl.BlockSpec((B, din, dout), lambda r, n: (0, 0, 0)),
            pl.BlockSpec((din, dout), lambda r, n: (0, 0)),
            pl.BlockSpec((1, dout), lambda r, n: (0, 0)),
        ],
        out_specs=pl.BlockSpec((1, _TN, dout), lambda r, n: (r, n, 0)),
        out_shape=jax.ShapeDtypeStruct((R + 1, N, dout), jnp.float32),
    )(comp_pad, x, basis, root, bias)


def _tc_inv(cnt2):
    def body(c_ref, o_ref):
        tot = c_ref[0] + c_ref[1]
        o_ref[...] = 1.0 / jnp.maximum(tot, 1.0)

    return pl.pallas_call(
        body,
        out_shape=jax.ShapeDtypeStruct((RN // 128, 128), jnp.float32),
    )(cnt2)


def _tc_combine(base, parts, relu):
    dout = base.shape[1]

    def body(b_ref, p_ref, o_ref):
        tot = b_ref[...] + p_ref[0] + p_ref[1]
        if relu:
            tot = jnp.maximum(tot, 0.0)
        o_ref[...] = tot

    return pl.pallas_call(
        body,
        grid=(N // _TN,),
        in_specs=[
            pl.BlockSpec((_TN, dout), lambda n: (n, 0)),
            pl.BlockSpec((NC, _TN, dout), lambda n: (0, n, 0)),
        ],
        out_specs=pl.BlockSpec((_TN, dout), lambda n: (n, 0)),
        out_shape=jax.ShapeDtypeStruct((N, dout), jnp.float32),
    )(base, parts)


def kernel(x, edge_index, edge_type, comp1, basis1, root1, bias1,
           comp2, basis2, root2, bias2):
    src = edge_index[0].astype(jnp.int32)
    dst = edge_index[1].astype(jnp.int32)
    et = edge_type.astype(jnp.int32)
    sidx = et * N + src
    cidx = et * N + dst

    comp1p = jnp.concatenate([comp1, jnp.zeros((1, B), comp1.dtype)], axis=0)
    comp2p = jnp.concatenate([comp2, jnp.zeros((1, B), comp2.dtype)], axis=0)

    cnt = _sc_counts(cidx)
    inv = _tc_inv(cnt.reshape(NC, RN // 128, 128)).reshape(RN)

    hx1 = _tc_dense(x, comp1p, basis1, root1, bias1.reshape(1, HID), HID)
    p1 = _sc_edge_pass(hx1.reshape((R + 1) * N, HID), inv, sidx, cidx, dst, HID)
    h = _tc_combine(hx1[R], p1, relu=True)

    hx2 = _tc_dense(h, comp2p, basis2, root2, bias2.reshape(1, OUT_CH), OUT_CH)
    p2 = _sc_edge_pass(hx2.reshape((R + 1) * N, OUT_CH), inv, sidx, cidx, dst,
                       OUT_CH)
    out = _tc_combine(hx2[R], p2, relu=False)
    return out


# speedup vs baseline: 46.1042x; 2.0532x over previous
import functools

import jax
import jax.numpy as jnp
from jax import lax
from jax.experimental import pallas as pl
from jax.experimental.pallas import tpu as pltpu
from jax.experimental.pallas import tpu_sc as plsc

N = 10000
E = 320000
IN_CH = 128
HID = 64
OUT_CH = 128
R = 8
B = 4

NC = 2
NS = 16
NW = NC * NS
EW = E // NW
K = 80
NB = EW // K
RN = R * N
CNT_SL = RN // NS
ROW_SL = N // NS


def _sc_mesh():
    return plsc.VectorSubcoreMesh(
        core_axis_name="c", subcore_axis_name="s", num_cores=NC, num_subcores=NS
    )


def _sc_counts(cidx):
    BNC = 5008

    @functools.partial(
        pl.kernel,
        out_type=jax.ShapeDtypeStruct((NC * RN,), jnp.float32),
        mesh=_sc_mesh(),
        scratch_types=[
            pltpu.VMEM((K,), jnp.int32),
            pltpu.VMEM((K,), jnp.float32),
            pltpu.VMEM((BNC,), jnp.float32),
            pltpu.VMEM_SHARED((RN,), jnp.float32),
        ],
    )
    def run(cidx_hbm, out_hbm, idx_v, ones_v, bnc_v, acc_sh):
        c = lax.axis_index("c")
        s = lax.axis_index("s")
        wid = s * NC + c

        def fill_ones(i, carry):
            ones_v[pl.ds(16 * i, 16)] = jnp.ones((16,), jnp.float32)
            return carry

        lax.fori_loop(0, K // 16, fill_ones, 0)

        def fill_z(i, carry):
            bnc_v[pl.ds(16 * i, 16)] = jnp.zeros((16,), jnp.float32)
            return carry

        lax.fori_loop(0, BNC // 16, fill_z, 0)
        pltpu.sync_copy(bnc_v.at[pl.ds(0, CNT_SL)],
                        acc_sh.at[pl.ds(s * CNT_SL, CNT_SL)])
        plsc.subcore_barrier()

        def blk(b, carry):
            pltpu.sync_copy(cidx_hbm.at[pl.ds(wid * EW + b * K, K)], idx_v)
            pltpu.sync_copy(ones_v, acc_sh.at[idx_v], add=True)
            return carry

        lax.fori_loop(0, NB, blk, 0)
        plsc.subcore_barrier()
        pltpu.sync_copy(acc_sh.at[pl.ds(s * CNT_SL, CNT_SL)],
                        bnc_v.at[pl.ds(0, CNT_SL)])
        pltpu.sync_copy(bnc_v.at[pl.ds(0, CNT_SL)],
                        out_hbm.at[pl.ds(c * RN + s * CNT_SL, CNT_SL)])

    return run(cidx).reshape(NC, RN)


def _sc_edge_pass(hx_flat, inv, sidx, cidx, dst, dout):
    j16 = dout // 16
    CH = 125
    D = 3

    @functools.partial(
        pl.kernel,
        out_type=jax.ShapeDtypeStruct((NC * N, dout), jnp.float32),
        mesh=_sc_mesh(),
        scratch_types=[
            [pltpu.VMEM((K,), jnp.int32) for _ in range(D)],
            [pltpu.VMEM((K,), jnp.int32) for _ in range(D)],
            [pltpu.VMEM((K,), jnp.int32) for _ in range(D)],
            [pltpu.VMEM((K,), jnp.float32) for _ in range(D)],
            [pltpu.VMEM((K, dout), jnp.float32) for _ in range(D)],
            pltpu.VMEM((CH, dout), jnp.float32),
            pltpu.VMEM_SHARED((N, dout), jnp.float32),
            [pltpu.SemaphoreType.DMA for _ in range(D)],
            [pltpu.SemaphoreType.DMA for _ in range(D)],
            [pltpu.SemaphoreType.DMA for _ in range(D)],
            [pltpu.SemaphoreType.DMA for _ in range(D)],
            [pltpu.SemaphoreType.DMA for _ in range(D)],
        ],
        compiler_params=pltpu.CompilerParams(
            use_tc_tiling_on_sc=False, needs_layout_passes=False),
    )
    def run(hx_hbm, inv_hbm, sidx_hbm, cidx_hbm, dst_hbm, out_hbm,
            sidx_v, cidx_v, dst_v, w_v, rows_v, bnc_v, acc_sh,
            sem_s, sem_t, sem_r, sem_w, sem_d):
        c = lax.axis_index("c")
        s = lax.axis_index("s")
        wid = s * NC + c
        NG = (NB + D - 1) // D

        def fill_z(i, carry):
            for j in range(j16):
                bnc_v[i, pl.ds(16 * j, 16)] = jnp.zeros((16,), jnp.float32)
            return carry

        lax.fori_loop(0, CH, fill_z, 0)

        def zero_chunk(q, carry):
            pltpu.sync_copy(bnc_v, acc_sh.at[pl.ds(s * ROW_SL + q * CH, CH)])
            return carry

        lax.fori_loop(0, ROW_SL // CH, zero_chunk, 0)
        plsc.subcore_barrier()

        def fire_sc(g, i):
            base = wid * EW + (g * D + i) * K
            pltpu.async_copy(sidx_hbm.at[pl.ds(base, K)], sidx_v[i], sem_s[i])
            pltpu.async_copy(cidx_hbm.at[pl.ds(base, K)], cidx_v[i], sem_s[i])

        for i in range(D):
            fire_sc(0, i)

        def group(g, carry):
            for i in range(D):
                live = g * D + i < NB

                @pl.when(jnp.logical_and(live, g > 0))
                def _(i=i):
                    pltpu.make_async_copy(hx_hbm.at[pl.ds(0, K)], rows_v[i],
                                          sem_d[i]).wait()

                @pl.when(live)
                def _(g=g, i=i):
                    base = wid * EW + (g * D + i) * K
                    pltpu.async_copy(dst_hbm.at[pl.ds(base, K)], dst_v[i],
                                     sem_t[i])
                    base0 = wid * EW
                    pltpu.make_async_copy(sidx_hbm.at[pl.ds(base0, K)],
                                          sidx_v[i], sem_s[i]).wait()
                    pltpu.make_async_copy(cidx_hbm.at[pl.ds(base0, K)],
                                          cidx_v[i], sem_s[i]).wait()
                    pltpu.async_copy(hx_hbm.at[sidx_v[i]], rows_v[i],
                                     sem_r[i])
                    pltpu.async_copy(inv_hbm.at[cidx_v[i]], w_v[i], sem_w[i])

            for i in range(D):
                live = g * D + i < NB

                @pl.when(live)
                def _(g=g, i=i):
                    rv = rows_v[i]
                    wv = w_v[i]
                    pltpu.make_async_copy(hx_hbm.at[pl.ds(0, K)], rv,
                                          sem_r[i]).wait()
                    pltpu.make_async_copy(inv_hbm.at[pl.ds(0, K)], wv,
                                          sem_w[i]).wait()

                    def edge(e, carry2):
                        w = plsc.load_gather(
                            wv, [jnp.full((16,), e, jnp.int32)])
                        for j in range(j16):
                            rv[e, pl.ds(16 * j, 16)] = (
                                rv[e, pl.ds(16 * j, 16)] * w)
                        return carry2

                    lax.fori_loop(0, K, edge, 0, unroll=8)
                    base = wid * EW
                    pltpu.make_async_copy(dst_hbm.at[pl.ds(base, K)],
                                          dst_v[i], sem_t[i]).wait()
                    pltpu.async_copy(rv, acc_sh.at[dst_v[i]], sem_d[i],
                                     add=True)

                @pl.when((g + 1) * D + i < NB)
                def _(g=g, i=i):
                    fire_sc(g + 1, i)

            return carry

        lax.fori_loop(0, NG, group, 0)
        for i in range(D):
            pltpu.make_async_copy(hx_hbm.at[pl.ds(0, K)], rows_v[i],
                                  sem_d[i]).wait()
        plsc.subcore_barrier()

        def out_chunk(q, carry):
            pltpu.sync_copy(acc_sh.at[pl.ds(s * ROW_SL + q * CH, CH)], bnc_v)
            pltpu.sync_copy(
                bnc_v, out_hbm.at[pl.ds(c * N + s * ROW_SL + q * CH, CH)])
            return carry

        lax.fori_loop(0, ROW_SL // CH, out_chunk, 0)

    return run(hx_flat, inv, sidx, cidx, dst).reshape(NC, N, dout)


_TN = 2000


def _tc_dense(x, comp_pad, basis, root, bias, dout):
    din = x.shape[1]

    def body(comp_ref, x_ref, basis_ref, root_ref, bias_ref, out_ref):
        r = pl.program_id(0)
        w = jnp.zeros((din, dout), jnp.float32)
        for b in range(B):
            w = w + comp_ref[r, b] * basis_ref[b]
        is_root = jnp.where(r == R, 1.0, 0.0).astype(jnp.float32)
        w = w + is_root * root_ref[...]
        acc = jnp.dot(x_ref[...], w, preferred_element_type=jnp.float32)
        out_ref[0] = acc + is_root * bias_ref[...]

    return pl.pallas_call(
        body,
        grid=(R + 1, N // _TN),
        in_specs=[
            pl.BlockSpec(memory_space=pltpu.SMEM),
            pl.BlockSpec((_TN, din), lambda r, n: (n, 0)),
            pl.BlockSpec((B, din, dout), lambda r, n: (0, 0, 0)),
            pl.BlockSpec((din, dout), lambda r, n: (0, 0)),
            pl.BlockSpec((1, dout), lambda r, n: (0, 0)),
        ],
        out_specs=pl.BlockSpec((1, _TN, dout), lambda r, n: (r, n, 0)),
        out_shape=jax.ShapeDtypeStruct((R + 1, N, dout), jnp.float32),
    )(comp_pad, x, basis, root, bias)


def _tc_inv(cnt2):
    def body(c_ref, o_ref):
        tot = c_ref[0] + c_ref[1]
        o_ref[...] = 1.0 / jnp.maximum(tot, 1.0)

    return pl.pallas_call(
        body,
        out_shape=jax.ShapeDtypeStruct((RN // 128, 128), jnp.float32),
    )(cnt2)


def _tc_combine(base, parts, relu):
    dout = base.shape[1]

    def body(b_ref, p_ref, o_ref):
        tot = b_ref[...] + p_ref[0] + p_ref[1]
        if relu:
            tot = jnp.maximum(tot, 0.0)
        o_ref[...] = tot

    return pl.pallas_call(
        body,
        grid=(N // _TN,),
        in_specs=[
            pl.BlockSpec((_TN, dout), lambda n: (n, 0)),
            pl.BlockSpec((NC, _TN, dout), lambda n: (0, n, 0)),
        ],
        out_specs=pl.BlockSpec((_TN, dout), lambda n: (n, 0)),
        out_shape=jax.ShapeDtypeStruct((N, dout), jnp.float32),
    )(base, parts)


def kernel(x, edge_index, edge_type, comp1, basis1, root1, bias1,
           comp2, basis2, root2, bias2):
    src = edge_index[0].astype(jnp.int32)
    dst = edge_index[1].astype(jnp.int32)
    et = edge_type.astype(jnp.int32)
    sidx = et * N + src
    cidx = et * N + dst

    comp1p = jnp.concatenate([comp1, jnp.zeros((1, B), comp1.dtype)], axis=0)
    comp2p = jnp.concatenate([comp2, jnp.zeros((1, B), comp2.dtype)], axis=0)

    cnt = _sc_counts(cidx)
    inv = _tc_inv(cnt.reshape(NC, RN // 128, 128)).reshape(RN)

    hx1 = _tc_dense(x, comp1p, basis1, root1, bias1.reshape(1, HID), HID)
    p1 = _sc_edge_pass(hx1.reshape((R + 1) * N, HID), inv, sidx, cidx, dst, HID)
    h = _tc_combine(hx1[R], p1, relu=True)

    hx2 = _tc_dense(h, comp2p, basis2, root2, bias2.reshape(1, OUT_CH), OUT_CH)
    p2 = _sc_edge_pass(hx2.reshape((R + 1) * N, OUT_CH), inv, sidx, cidx, dst,
                       OUT_CH)
    out = _tc_combine(hx2[R], p2, relu=False)
    return out


# speedup vs baseline: 46.2427x; 1.0030x over previous
import functools

import jax
import jax.numpy as jnp
from jax import lax
from jax.experimental import pallas as pl
from jax.experimental.pallas import tpu as pltpu
from jax.experimental.pallas import tpu_sc as plsc

N = 10000
E = 320000
IN_CH = 128
HID = 64
OUT_CH = 128
R = 8
B = 4

NC = 2
NS = 16
NW = NC * NS
EW = E // NW
K = 80
NB = EW // K
RN = R * N
CNT_SL = RN // NS
ROW_SL = N // NS


def _sc_mesh():
    return plsc.VectorSubcoreMesh(
        core_axis_name="c", subcore_axis_name="s", num_cores=NC, num_subcores=NS
    )


def _sc_counts(cidx):
    BNC = 5008
    D = 3

    @functools.partial(
        pl.kernel,
        out_type=jax.ShapeDtypeStruct((NC * RN,), jnp.float32),
        mesh=_sc_mesh(),
        scratch_types=[
            [pltpu.VMEM((K,), jnp.int32) for _ in range(D)],
            pltpu.VMEM((K,), jnp.float32),
            pltpu.VMEM((BNC,), jnp.float32),
            pltpu.VMEM_SHARED((RN,), jnp.float32),
            [pltpu.SemaphoreType.DMA for _ in range(D)],
            [pltpu.SemaphoreType.DMA for _ in range(D)],
        ],
    )
    def run(cidx_hbm, out_hbm, idx_v, ones_v, bnc_v, acc_sh, sem_a, sem_d):
        c = lax.axis_index("c")
        s = lax.axis_index("s")
        wid = s * NC + c
        NG = (NB + D - 1) // D

        def fill_ones(i, carry):
            ones_v[pl.ds(16 * i, 16)] = jnp.ones((16,), jnp.float32)
            return carry

        lax.fori_loop(0, K // 16, fill_ones, 0)

        def fill_z(i, carry):
            bnc_v[pl.ds(16 * i, 16)] = jnp.zeros((16,), jnp.float32)
            return carry

        lax.fori_loop(0, BNC // 16, fill_z, 0)
        pltpu.sync_copy(bnc_v.at[pl.ds(0, CNT_SL)],
                        acc_sh.at[pl.ds(s * CNT_SL, CNT_SL)])
        plsc.subcore_barrier()

        for i in range(D):
            pltpu.async_copy(cidx_hbm.at[pl.ds(wid * EW + i * K, K)],
                             idx_v[i], sem_a[i])

        def group(g, carry):
            for i in range(D):
                live = g * D + i < NB

                @pl.when(live)
                def _(g=g, i=i):
                    pltpu.make_async_copy(cidx_hbm.at[pl.ds(wid * EW, K)],
                                          idx_v[i], sem_a[i]).wait()
                    pltpu.async_copy(ones_v, acc_sh.at[idx_v[i]], sem_d[i],
                                     add=True)

                @pl.when(jnp.logical_and(live, (g + 1) * D + i < NB))
                def _(g=g, i=i):
                    pltpu.make_async_copy(ones_v, acc_sh.at[pl.ds(0, K)],
                                          sem_d[i]).wait()
                    base = wid * EW + ((g + 1) * D + i) * K
                    pltpu.async_copy(cidx_hbm.at[pl.ds(base, K)], idx_v[i],
                                     sem_a[i])

            return carry

        lax.fori_loop(0, NG, group, 0)
        for i in range(D):
            pltpu.make_async_copy(ones_v, acc_sh.at[pl.ds(0, K)],
                                  sem_d[i]).wait()
        plsc.subcore_barrier()
        pltpu.sync_copy(acc_sh.at[pl.ds(s * CNT_SL, CNT_SL)],
                        bnc_v.at[pl.ds(0, CNT_SL)])
        pltpu.sync_copy(bnc_v.at[pl.ds(0, CNT_SL)],
                        out_hbm.at[pl.ds(c * RN + s * CNT_SL, CNT_SL)])

    return run(cidx).reshape(NC, RN)


def _sc_edge_pass(hx_flat, inv, sidx, cidx, dst, dout):
    j16 = dout // 16
    CH = 125
    D = 3

    @functools.partial(
        pl.kernel,
        out_type=jax.ShapeDtypeStruct((NC * N, dout), jnp.float32),
        mesh=_sc_mesh(),
        scratch_types=[
            [pltpu.VMEM((K,), jnp.int32) for _ in range(D)],
            [pltpu.VMEM((K,), jnp.int32) for _ in range(D)],
            [pltpu.VMEM((K,), jnp.int32) for _ in range(D)],
            [pltpu.VMEM((K,), jnp.float32) for _ in range(D)],
            [pltpu.VMEM((K, dout), jnp.float32) for _ in range(D)],
            pltpu.VMEM((CH, dout), jnp.float32),
            pltpu.VMEM_SHARED((N, dout), jnp.float32),
            [pltpu.SemaphoreType.DMA for _ in range(D)],
            [pltpu.SemaphoreType.DMA for _ in range(D)],
            [pltpu.SemaphoreType.DMA for _ in range(D)],
            [pltpu.SemaphoreType.DMA for _ in range(D)],
            [pltpu.SemaphoreType.DMA for _ in range(D)],
        ],
        compiler_params=pltpu.CompilerParams(
            use_tc_tiling_on_sc=False, needs_layout_passes=False),
    )
    def run(hx_hbm, inv_hbm, sidx_hbm, cidx_hbm, dst_hbm, out_hbm,
            sidx_v, cidx_v, dst_v, w_v, rows_v, bnc_v, acc_sh,
            sem_s, sem_t, sem_r, sem_w, sem_d):
        c = lax.axis_index("c")
        s = lax.axis_index("s")
        wid = s * NC + c
        NG = (NB + D - 1) // D

        def fill_z(i, carry):
            for j in range(j16):
                bnc_v[i, pl.ds(16 * j, 16)] = jnp.zeros((16,), jnp.float32)
            return carry

        lax.fori_loop(0, CH, fill_z, 0)

        def zero_chunk(q, carry):
            pltpu.sync_copy(bnc_v, acc_sh.at[pl.ds(s * ROW_SL + q * CH, CH)])
            return carry

        lax.fori_loop(0, ROW_SL // CH, zero_chunk, 0)
        plsc.subcore_barrier()

        def fire_sc(g, i):
            base = wid * EW + (g * D + i) * K
            pltpu.async_copy(sidx_hbm.at[pl.ds(base, K)], sidx_v[i], sem_s[i])
            pltpu.async_copy(cidx_hbm.at[pl.ds(base, K)], cidx_v[i], sem_s[i])

        for i in range(D):
            fire_sc(0, i)

        def group(g, carry):
            for i in range(D):
                live = g * D + i < NB

                @pl.when(jnp.logical_and(live, g > 0))
                def _(i=i):
                    pltpu.make_async_copy(hx_hbm.at[pl.ds(0, K)], rows_v[i],
                                          sem_d[i]).wait()

                @pl.when(live)
                def _(g=g, i=i):
                    base = wid * EW + (g * D + i) * K
                    pltpu.async_copy(dst_hbm.at[pl.ds(base, K)], dst_v[i],
                                     sem_t[i])
                    base0 = wid * EW
                    pltpu.make_async_copy(sidx_hbm.at[pl.ds(base0, K)],
                                          sidx_v[i], sem_s[i]).wait()
                    pltpu.make_async_copy(cidx_hbm.at[pl.ds(base0, K)],
                                          cidx_v[i], sem_s[i]).wait()
                    pltpu.async_copy(hx_hbm.at[sidx_v[i]], rows_v[i],
                                     sem_r[i])
                    pltpu.async_copy(inv_hbm.at[cidx_v[i]], w_v[i], sem_w[i])

            for i in range(D):
                live = g * D + i < NB

                @pl.when(live)
                def _(g=g, i=i):
                    rv = rows_v[i]
                    wv = w_v[i]
                    pltpu.make_async_copy(hx_hbm.at[pl.ds(0, K)], rv,
                                          sem_r[i]).wait()
                    pltpu.make_async_copy(inv_hbm.at[pl.ds(0, K)], wv,
                                          sem_w[i]).wait()

                    def edge(e, carry2):
                        w = plsc.load_gather(
                            wv, [jnp.full((16,), e, jnp.int32)])
                        for j in range(j16):
                            rv[e, pl.ds(16 * j, 16)] = (
                                rv[e, pl.ds(16 * j, 16)] * w)
                        return carry2

                    lax.fori_loop(0, K, edge, 0, unroll=8)
                    base = wid * EW
                    pltpu.make_async_copy(dst_hbm.at[pl.ds(base, K)],
                                          dst_v[i], sem_t[i]).wait()
                    pltpu.async_copy(rv, acc_sh.at[dst_v[i]], sem_d[i],
                                     add=True)

                @pl.when((g + 1) * D + i < NB)
                def _(g=g, i=i):
                    fire_sc(g + 1, i)

            return carry

        lax.fori_loop(0, NG, group, 0)
        for i in range(D):
            pltpu.make_async_copy(hx_hbm.at[pl.ds(0, K)], rows_v[i],
                                  sem_d[i]).wait()
        plsc.subcore_barrier()

        def out_chunk(q, carry):
            pltpu.sync_copy(acc_sh.at[pl.ds(s * ROW_SL + q * CH, CH)], bnc_v)
            pltpu.sync_copy(
                bnc_v, out_hbm.at[pl.ds(c * N + s * ROW_SL + q * CH, CH)])
            return carry

        lax.fori_loop(0, ROW_SL // CH, out_chunk, 0)

    return run(hx_flat, inv, sidx, cidx, dst).reshape(NC, N, dout)


_TN = 2000


def _tc_dense(x, comp_pad, basis, root, bias, dout):
    din = x.shape[1]

    def body(comp_ref, x_ref, basis_ref, root_ref, bias_ref, out_ref):
        r = pl.program_id(0)
        w = jnp.zeros((din, dout), jnp.float32)
        for b in range(B):
            w = w + comp_ref[r, b] * basis_ref[b]
        is_root = jnp.where(r == R, 1.0, 0.0).astype(jnp.float32)
        w = w + is_root * root_ref[...]
        acc = jnp.dot(x_ref[...], w, preferred_element_type=jnp.float32)
        out_ref[0] = acc + is_root * bias_ref[...]

    return pl.pallas_call(
        body,
        grid=(R + 1, N // _TN),
        in_specs=[
            pl.BlockSpec(memory_space=pltpu.SMEM),
            pl.BlockSpec((_TN, din), lambda r, n: (n, 0)),
            pl.BlockSpec((B, din, dout), lambda r, n: (0, 0, 0)),
            pl.BlockSpec((din, dout), lambda r, n: (0, 0)),
            pl.BlockSpec((1, dout), lambda r, n: (0, 0)),
        ],
        out_specs=pl.BlockSpec((1, _TN, dout), lambda r, n: (r, n, 0)),
        out_shape=jax.ShapeDtypeStruct((R + 1, N, dout), jnp.float32),
    )(comp_pad, x, basis, root, bias)


def _tc_inv(cnt2):
    def body(c_ref, o_ref):
        tot = c_ref[0] + c_ref[1]
        o_ref[...] = 1.0 / jnp.maximum(tot, 1.0)

    return pl.pallas_call(
        body,
        out_shape=jax.ShapeDtypeStruct((RN // 128, 128), jnp.float32),
    )(cnt2)


def _tc_combine(base, parts, relu):
    dout = base.shape[1]

    def body(b_ref, p_ref, o_ref):
        tot = b_ref[...] + p_ref[0] + p_ref[1]
        if relu:
            tot = jnp.maximum(tot, 0.0)
        o_ref[...] = tot

    return pl.pallas_call(
        body,
        grid=(N // _TN,),
        in_specs=[
            pl.BlockSpec((_TN, dout), lambda n: (n, 0)),
            pl.BlockSpec((NC, _TN, dout), lambda n: (0, n, 0)),
        ],
        out_specs=pl.BlockSpec((_TN, dout), lambda n: (n, 0)),
        out_shape=jax.ShapeDtypeStruct((N, dout), jnp.float32),
    )(base, parts)


def kernel(x, edge_index, edge_type, comp1, basis1, root1, bias1,
           comp2, basis2, root2, bias2):
    src = edge_index[0].astype(jnp.int32)
    dst = edge_index[1].astype(jnp.int32)
    et = edge_type.astype(jnp.int32)
    sidx = et * N + src
    cidx = et * N + dst

    comp1p = jnp.concatenate([comp1, jnp.zeros((1, B), comp1.dtype)], axis=0)
    comp2p = jnp.concatenate([comp2, jnp.zeros((1, B), comp2.dtype)], axis=0)

    cnt = _sc_counts(cidx)
    inv = _tc_inv(cnt.reshape(NC, RN // 128, 128)).reshape(RN)

    hx1 = _tc_dense(x, comp1p, basis1, root1, bias1.reshape(1, HID), HID)
    p1 = _sc_edge_pass(hx1.reshape((R + 1) * N, HID), inv, sidx, cidx, dst, HID)
    h = _tc_combine(hx1[R], p1, relu=True)

    hx2 = _tc_dense(h, comp2p, basis2, root2, bias2.reshape(1, OUT_CH), OUT_CH)
    p2 = _sc_edge_pass(hx2.reshape((R + 1) * N, OUT_CH), inv, sidx, cidx, dst,
                       OUT_CH)
    out = _tc_combine(hx2[R], p2, relu=False)
    return out
